# baseline (speedup 1.0000x reference)
import functools

import jax
import jax.numpy as jnp
from jax import lax
from jax.experimental import pallas as pl
from jax.experimental.pallas import tpu as pltpu
from jax.experimental.pallas import tpu_sc as plsc

_N, _A, _C = 32, 8732, 21
_APAD = 8736
_NCH = _APAD // 16
_CTW = 8832
_LW = _APAD * 4


def _ce_body(cp_ref, ct_ref,
             negce_ref, negbits_ref, pvec_ref, pcnt_ref, posce_ref):
    i = pl.program_id(0)
    x = cp_ref[0]
    tgt = ct_ref[0]

    lidx = jax.lax.broadcasted_iota(jnp.int32, (1, _A), 1)
    valid = lidx < _A
    pos = (tgt > 0) & valid

    m = jnp.max(x, axis=0, keepdims=True)
    s = jnp.sum(jnp.exp(x - m), axis=0, keepdims=True)
    lse = m + jnp.log(s)
    cidx = jax.lax.broadcasted_iota(jnp.int32, (_C, _A), 0)
    xt = jnp.sum(jnp.where(cidx == tgt, x, 0.0), axis=0, keepdims=True)
    ce = lse - xt

    negce = jnp.concatenate(
        [jnp.where(valid & (tgt == 0), ce, -1.0),
         jnp.full((1, _APAD - _A), -1.0, jnp.float32)], axis=1)
    negce_ref[0] = negce
    negbits_ref[0] = jax.lax.bitcast_convert_type(negce, jnp.int32)

    pce = jnp.sum(jnp.where(pos, ce, 0.0))
    pc = jnp.sum(jnp.where(pos, 1, 0).astype(jnp.int32))

    pcnt_ref[i] = pc
    pvec_ref[0] = jnp.full((1, 16), pc, jnp.int32).astype(jnp.float32)

    @pl.when(i == 0)
    def _():
        posce_ref[0] = pce

    @pl.when(i != 0)
    def _():
        posce_ref[0] += pce


def _splat_sum(x):
    for d in (1, 2, 4, 8):
        idx = lax.iota(jnp.int32, 16) ^ d
        x = x + x.at[idx].get(mode="promise_in_bounds")
    return x


def _splat_max(x):
    for d in (1, 2, 4, 8):
        idx = lax.iota(jnp.int32, 16) ^ d
        x = jnp.maximum(x, x.at[idx].get(mode="promise_in_bounds"))
    return x


def _sc_loc_body(lp_hbm, lt_hbm, ct_hbm, out_hbm, lp_v, lt_v, ct_v, out_v):
    wid = lax.axis_index("s") * 2 + lax.axis_index("c")
    pltpu.sync_copy(lp_hbm.at[wid], lp_v)
    pltpu.sync_copy(lt_hbm.at[wid], lt_v)
    pltpu.sync_copy(ct_hbm.at[wid], ct_v)

    zf = jnp.zeros((16,), jnp.float32)
    lane4 = lax.shift_right_logical(lax.iota(jnp.int32, 16), 2)

    def body(q, acc):
        tgt16 = ct_v[pl.ds(q * 16, 16)]
        for j in range(4):
            vp = lp_v[pl.ds((q * 4 + j) * 16, 16)]
            vt = lt_v[pl.ds((q * 4 + j) * 16, 16)]
            t4 = tgt16.at[j * 4 + lane4].get(mode="promise_in_bounds")
            d = vp - vt
            ad = jnp.abs(d)
            sl1 = jnp.where(ad < 1.0, 0.5 * d * d, ad - 0.5)
            acc = acc + jnp.where(t4 > 0, sl1, zf)
        return acc

    acc = lax.fori_loop(0, _NCH, body, zf)
    out_v[...] = _splat_sum(acc)
    pltpu.sync_copy(out_v, out_hbm.at[wid])


def _sc_mine_body(negce_hbm, bits_hbm, pp_hbm, out_hbm, row_v, row_iv, p_v,
                  out_v):
    wid = lax.axis_index("s") * 2 + lax.axis_index("c")
    pltpu.sync_copy(negce_hbm.at[wid], row_v)
    pltpu.sync_copy(bits_hbm.at[wid], row_iv)
    pltpu.sync_copy(pp_hbm.at[wid], p_v)
    p = p_v[...][0]
    k = jnp.minimum(3.0 * p, _A - p)

    zf = jnp.zeros((16,), jnp.float32)
    onef = jnp.ones((16,), jnp.float32)

    def scalarize(vec):
        return vec[0]

    def pass1(c, carry):
        s, n = carry
        v = row_v[pl.ds(c * 16, 16)]
        m = v >= 0.0
        return s + jnp.where(m, v, zf), n + jnp.where(m, onef, zf)

    s, n = lax.fori_loop(0, _NCH, pass1, (zf, zf))
    s_all = scalarize(_splat_sum(s))
    nneg = scalarize(_splat_sum(n))

    def fast(_):
        return s_all

    def slow(_):
        def bitstep(b, t):
            cand = t | jnp.left_shift(jnp.int32(1), 30 - b)

            def cnt_body(c, acc):
                bits = row_iv[pl.ds(c * 16, 16)]
                return acc + jnp.where(bits >= cand, onef, zf)

            cnt = scalarize(_splat_sum(lax.fori_loop(0, _NCH, cnt_body, zf)))
            return jnp.where(cnt >= k, cand, t)

        t = lax.fori_loop(0, 31, bitstep, jnp.int32(0))

        def fin(c, carry):
            sg, cg, te = carry
            v = row_v[pl.ds(c * 16, 16)]
            bits = row_iv[pl.ds(c * 16, 16)]
            m = bits > t
            return (sg + jnp.where(m, v, zf), cg + jnp.where(m, onef, zf),
                    jnp.where(bits == t, v, te))

        sg, cg, te = lax.fori_loop(0, _NCH, fin, (zf, zf, zf))
        sgt = scalarize(_splat_sum(sg))
        cgt = scalarize(_splat_sum(cg))
        tf = scalarize(_splat_max(te))
        mult = jnp.maximum(k - cgt, 0.0)
        return sgt + mult * tf

    res = lax.cond(k >= nneg, fast, slow, 0)
    out_v[...] = jnp.full((16,), res, jnp.float32)
    pltpu.sync_copy(out_v, out_hbm.at[wid])


@jax.jit
def kernel(loc_preds, loc_targets, cls_preds, cls_targets):
    cpT = jnp.transpose(cls_preds, (0, 2, 1))
    ct3 = cls_targets.reshape(_N, 1, _A)
    ct_pad = jnp.pad(cls_targets, ((0, 0), (0, _CTW - _A)))
    lp_pad = jnp.pad(loc_preds.reshape(_N, _A * 4),
                     ((0, 0), (0, _LW - _A * 4)))
    lt_pad = jnp.pad(loc_targets.reshape(_N, _A * 4),
                     ((0, 0), (0, _LW - _A * 4)))

    sc_loc = pl.kernel(
        _sc_loc_body,
        out_type=jax.ShapeDtypeStruct((_N, 16), jnp.float32),
        mesh=plsc.VectorSubcoreMesh(core_axis_name="c", subcore_axis_name="s"),
        scratch_types=[
            pltpu.VMEM((_LW,), jnp.float32),
            pltpu.VMEM((_LW,), jnp.float32),
            pltpu.VMEM((_CTW,), jnp.int32),
            pltpu.VMEM((16,), jnp.float32),
        ],
    )
    loc_rows = sc_loc(lp_pad, lt_pad, ct_pad)

    negce3, negbits3, pvec3, pcnt, posce = pl.pallas_call(
        _ce_body,
        grid=(_N,),
        in_specs=[
            pl.BlockSpec((1, _C, _A), lambda i: (i, 0, 0)),
            pl.BlockSpec((1, 1, _A), lambda i: (i, 0, 0)),
        ],
        out_specs=[
            pl.BlockSpec((1, 1, _APAD), lambda i: (i, 0, 0)),
            pl.BlockSpec((1, 1, _APAD), lambda i: (i, 0, 0)),
            pl.BlockSpec((1, 1, 16), lambda i: (i, 0, 0)),
            pl.BlockSpec(memory_space=pltpu.SMEM),
            pl.BlockSpec(memory_space=pltpu.SMEM),
        ],
        out_shape=[
            jax.ShapeDtypeStruct((_N, 1, _APAD), jnp.float32),
            jax.ShapeDtypeStruct((_N, 1, _APAD), jnp.int32),
            jax.ShapeDtypeStruct((_N, 1, 16), jnp.float32),
            jax.ShapeDtypeStruct((_N,), jnp.int32),
            jax.ShapeDtypeStruct((1,), jnp.float32),
        ],
    )(cpT, ct3)

    sc_mine = pl.kernel(
        _sc_mine_body,
        out_type=jax.ShapeDtypeStruct((_N, 16), jnp.float32),
        mesh=plsc.VectorSubcoreMesh(core_axis_name="c", subcore_axis_name="s"),
        scratch_types=[
            pltpu.VMEM((_APAD,), jnp.float32),
            pltpu.VMEM((_APAD,), jnp.int32),
            pltpu.VMEM((16,), jnp.float32),
            pltpu.VMEM((16,), jnp.float32),
        ],
    )
    topk_rows = sc_mine(negce3.reshape(_N, _APAD),
                        negbits3.reshape(_N, _APAD),
                        pvec3.reshape(_N, 16))

    num_pos = jnp.sum(pcnt).astype(jnp.float32)
    cls_loss = posce[0] + jnp.sum(topk_rows[:, 0])
    loc_loss = jnp.sum(loc_rows[:, 0])
    return (loc_loss / num_pos, cls_loss / num_pos)


# speedup vs baseline: 1.4423x; 1.4423x over previous
import jax
import jax.numpy as jnp
from jax import lax
from jax.experimental import pallas as pl
from jax.experimental.pallas import tpu as pltpu
from jax.experimental.pallas import tpu_sc as plsc

_N, _A, _C = 32, 8732, 21
_APAD = 8736
_NCH = _APAD // 16


def _ce_loc_body(cp_ref, ct_ref, lp_ref, lt_ref,
                 negce_ref, negbits_ref, pvec_ref, pcnt_ref, posce_ref,
                 locsum_ref):
    i = pl.program_id(0)
    x = cp_ref[0]
    tgt = ct_ref[0]
    lp = lp_ref[0]
    lt = lt_ref[0]

    lidx = jax.lax.broadcasted_iota(jnp.int32, (1, _A), 1)
    valid = lidx < _A
    pos = (tgt > 0) & valid

    m = jnp.max(x, axis=0, keepdims=True)
    s = jnp.sum(jnp.exp(x - m), axis=0, keepdims=True)
    lse = m + jnp.log(s)
    cidx = jax.lax.broadcasted_iota(jnp.int32, (_C, _A), 0)
    xt = jnp.sum(jnp.where(cidx == tgt, x, 0.0), axis=0, keepdims=True)
    ce = lse - xt

    negce = jnp.concatenate(
        [jnp.where(valid & (tgt == 0), ce, -1.0),
         jnp.full((1, _APAD - _A), -1.0, jnp.float32)], axis=1)
    negce_ref[0] = negce
    negbits_ref[0] = jax.lax.bitcast_convert_type(negce, jnp.int32)

    d = lp - lt
    ad = jnp.abs(d)
    sl1 = jnp.sum(jnp.where(ad < 1.0, 0.5 * d * d, ad - 0.5),
                  axis=0, keepdims=True)
    lsum = jnp.sum(jnp.where(pos, sl1, 0.0))
    pce = jnp.sum(jnp.where(pos, ce, 0.0))
    pc = jnp.sum(jnp.where(pos, 1, 0).astype(jnp.int32))

    pcnt_ref[i] = pc
    pvec_ref[0] = jnp.full((1, 16), pc, jnp.int32).astype(jnp.float32)

    @pl.when(i == 0)
    def _():
        posce_ref[0] = pce
        locsum_ref[0] = lsum

    @pl.when(i != 0)
    def _():
        posce_ref[0] += pce
        locsum_ref[0] += lsum


def _splat_sum(x):
    for d in (1, 2, 4, 8):
        idx = lax.iota(jnp.int32, 16) ^ d
        x = x + x.at[idx].get(mode="promise_in_bounds")
    return x


def _splat_max(x):
    for d in (1, 2, 4, 8):
        idx = lax.iota(jnp.int32, 16) ^ d
        x = jnp.maximum(x, x.at[idx].get(mode="promise_in_bounds"))
    return x


def _sc_mine_body(negce_hbm, bits_hbm, pp_hbm, out_hbm, row_v, row_iv, p_v,
                  out_v):
    wid = lax.axis_index("s") * 2 + lax.axis_index("c")
    pltpu.sync_copy(negce_hbm.at[wid], row_v)
    pltpu.sync_copy(bits_hbm.at[wid], row_iv)
    pltpu.sync_copy(pp_hbm.at[wid], p_v)
    p = p_v[...][0]
    k = jnp.minimum(3.0 * p, _A - p)

    zf = jnp.zeros((16,), jnp.float32)
    onef = jnp.ones((16,), jnp.float32)

    def scalarize(vec):
        return vec[0]

    def pass1(c, carry):
        s, n = carry
        v = row_v[pl.ds(c * 16, 16)]
        m = v >= 0.0
        return s + jnp.where(m, v, zf), n + jnp.where(m, onef, zf)

    s, n = lax.fori_loop(0, _NCH, pass1, (zf, zf))
    s_all = scalarize(_splat_sum(s))
    nneg = scalarize(_splat_sum(n))

    def fast(_):
        return s_all

    def slow(_):
        def bitstep(b, t):
            cand = t | jnp.left_shift(jnp.int32(1), 30 - b)

            def cnt_body(c, acc):
                bits = row_iv[pl.ds(c * 16, 16)]
                return acc + jnp.where(bits >= cand, onef, zf)

            cnt = scalarize(_splat_sum(lax.fori_loop(0, _NCH, cnt_body, zf)))
            return jnp.where(cnt >= k, cand, t)

        t = lax.fori_loop(0, 31, bitstep, jnp.int32(0))

        def fin(c, carry):
            sg, cg, te = carry
            v = row_v[pl.ds(c * 16, 16)]
            bits = row_iv[pl.ds(c * 16, 16)]
            m = bits > t
            return (sg + jnp.where(m, v, zf), cg + jnp.where(m, onef, zf),
                    jnp.where(bits == t, v, te))

        sg, cg, te = lax.fori_loop(0, _NCH, fin, (zf, zf, zf))
        sgt = scalarize(_splat_sum(sg))
        cgt = scalarize(_splat_sum(cg))
        tf = scalarize(_splat_max(te))
        mult = jnp.maximum(k - cgt, 0.0)
        return sgt + mult * tf

    res = lax.cond(k >= nneg, fast, slow, 0)
    out_v[...] = jnp.full((16,), res, jnp.float32)
    pltpu.sync_copy(out_v, out_hbm.at[wid])


@jax.jit
def kernel(loc_preds, loc_targets, cls_preds, cls_targets):
    cpT = jnp.transpose(cls_preds, (0, 2, 1))
    lpT = jnp.transpose(loc_preds, (0, 2, 1))
    ltT = jnp.transpose(loc_targets, (0, 2, 1))
    ct3 = cls_targets.reshape(_N, 1, _A)

    negce3, negbits3, pvec3, pcnt, posce, locsum = pl.pallas_call(
        _ce_loc_body,
        grid=(_N,),
        in_specs=[
            pl.BlockSpec((1, _C, _A), lambda i: (i, 0, 0)),
            pl.BlockSpec((1, 1, _A), lambda i: (i, 0, 0)),
            pl.BlockSpec((1, 4, _A), lambda i: (i, 0, 0)),
            pl.BlockSpec((1, 4, _A), lambda i: (i, 0, 0)),
        ],
        out_specs=[
            pl.BlockSpec((1, 1, _APAD), lambda i: (i, 0, 0)),
            pl.BlockSpec((1, 1, _APAD), lambda i: (i, 0, 0)),
            pl.BlockSpec((1, 1, 16), lambda i: (i, 0, 0)),
            pl.BlockSpec(memory_space=pltpu.SMEM),
            pl.BlockSpec(memory_space=pltpu.SMEM),
            pl.BlockSpec(memory_space=pltpu.SMEM),
        ],
        out_shape=[
            jax.ShapeDtypeStruct((_N, 1, _APAD), jnp.float32),
            jax.ShapeDtypeStruct((_N, 1, _APAD), jnp.int32),
            jax.ShapeDtypeStruct((_N, 1, 16), jnp.float32),
            jax.ShapeDtypeStruct((_N,), jnp.int32),
            jax.ShapeDtypeStruct((1,), jnp.float32),
            jax.ShapeDtypeStruct((1,), jnp.float32),
        ],
    )(cpT, ct3, lpT, ltT)

    sc_mine = pl.kernel(
        _sc_mine_body,
        out_type=jax.ShapeDtypeStruct((_N, 16), jnp.float32),
        mesh=plsc.VectorSubcoreMesh(core_axis_name="c", subcore_axis_name="s"),
        scratch_types=[
            pltpu.VMEM((_APAD,), jnp.float32),
            pltpu.VMEM((_APAD,), jnp.int32),
            pltpu.VMEM((16,), jnp.float32),
            pltpu.VMEM((16,), jnp.float32),
        ],
    )
    topk_rows = sc_mine(negce3.reshape(_N, _APAD),
                        negbits3.reshape(_N, _APAD),
                        pvec3.reshape(_N, 16))

    num_pos = jnp.sum(pcnt).astype(jnp.float32)
    cls_loss = posce[0] + jnp.sum(topk_rows[:, 0])
    return (locsum[0] / num_pos, cls_loss / num_pos)
